# Initial kernel scaffold; baseline (speedup 1.0000x reference)
import jax
import jax.numpy as jnp
from jax.experimental import pallas as pl


def kernel(heatmap):
    raise NotImplementedError("write your pallas kernel here")


# speedup vs baseline: 1.3115x; 1.3115x over previous
import functools

import jax
import jax.numpy as jnp
from jax.experimental import pallas as pl

_TOPK = 8
_PAD = 2
_SUB = 8
_STRIP = 64


def _vshift(x, s, fill):
    h, w = x.shape
    if s == 0:
        return x
    pad = jnp.full((abs(s), w), fill, x.dtype)
    if s > 0:
        return jnp.concatenate([x[s:], pad], axis=0)
    return jnp.concatenate([pad, x[:h + s]], axis=0)


def _hshift(x, s, fill):
    h, w = x.shape
    if s == 0:
        return x
    pad = jnp.full((h, abs(s)), fill, x.dtype)
    if s > 0:
        return jnp.concatenate([x[:, s:], pad], axis=1)
    return jnp.concatenate([pad, x[:, :w + s]], axis=1)


def _hmax5(v):
    ninf = jnp.float32(-jnp.inf)
    p = v
    for s in (-2, -1, 1, 2):
        p = jnp.maximum(p, _hshift(v, s, ninf))
    return p


def _peaks_kernel(x_ref, score_ref, idx_ref):
    h, w = x_ref.shape[1], x_ref.shape[2]
    ninf = jnp.float32(-jnp.inf)
    big = jnp.int32(2**30)
    n_strips = h // _STRIP
    n_sub = _STRIP // _SUB

    smax = jnp.full((n_strips, n_sub), ninf, jnp.float32)
    i_strip = jax.lax.broadcasted_iota(jnp.int32, (n_strips, n_sub), 0)
    for s in range(n_strips):
        r0 = s * _STRIP
        lo = max(0, r0 - _PAD)
        hi = min(h, r0 + _STRIP + _PAD)
        xs = x_ref[0, lo:hi, :]
        top_fill = r0 - _PAD - lo
        bot_fill = (r0 + _STRIP + _PAD) - hi
        parts = []
        if top_fill < 0:
            parts.append(jnp.full((-top_fill, w), ninf, jnp.float32))
        parts.append(xs)
        if bot_fill > 0:
            parts.append(jnp.full((bot_fill, w), ninf, jnp.float32))
        xp = jnp.concatenate(parts, axis=0) if len(parts) > 1 else parts[0]
        v = xp[0:_STRIP]
        for d in range(1, 5):
            v = jnp.maximum(v, xp[d:d + _STRIP])
        p = _hmax5(v)
        xc = xp[_PAD:_PAD + _STRIP]
        m = jnp.where(xc == p, xc, jnp.float32(0.0))
        t = jnp.max(m.reshape(n_sub, _SUB, w), axis=1)
        srow = jnp.max(t, axis=1).reshape(1, n_sub)
        smax = jnp.where(i_strip == s, srow, smax)

    sub_iota = (i_strip * n_sub
                + jax.lax.broadcasted_iota(jnp.int32, (n_strips, n_sub), 1))
    lin_i0 = jax.lax.broadcasted_iota(jnp.int32, (_SUB, w), 0)
    lin_i1 = jax.lax.broadcasted_iota(jnp.int32, (_SUB, w), 1)
    vals = []
    idxs = []
    extracted = []
    for _ in range(_TOPK):
        gm = jnp.max(smax)
        si = jnp.min(jnp.where(smax == gm, sub_iota, big))
        r0 = si * _SUB
        rows = []
        for d in range(-_PAD, _SUB + _PAD):
            rr = r0 + d
            rc = jnp.clip(rr, 0, h - 1)
            xr = x_ref[0, pl.ds(rc, 1), :]
            valid = jnp.logical_and(rr >= 0, rr < h)
            rows.append(jnp.where(valid, xr, ninf))
        xp = jnp.concatenate(rows, axis=0)
        v = xp[0:_SUB]
        for d in range(1, 5):
            v = jnp.maximum(v, xp[d:d + _SUB])
        p = _hmax5(v)
        xc = xp[_PAD:_PAD + _SUB]
        m = jnp.where(xc == p, xc, jnp.float32(0.0))
        lin = (lin_i0 + r0) * w + lin_i1
        for prev in extracted:
            m = jnp.where(lin == prev, ninf, m)
        gi = jnp.min(jnp.where(m == gm, lin, big))
        extracted.append(gi)
        vals.append(gm)
        idxs.append(gi)
        m2 = jnp.where(lin == gi, ninf, m)
        smax = jnp.where(sub_iota == si, jnp.max(m2), smax)

    k_iota = jax.lax.broadcasted_iota(jnp.int32, (1, _TOPK), 1)
    vvec = jnp.zeros((1, _TOPK), jnp.float32)
    ivec = jnp.zeros((1, _TOPK), jnp.int32)
    for k in range(_TOPK):
        vvec = jnp.where(k_iota == k, vals[k], vvec)
        ivec = jnp.where(k_iota == k, idxs[k], ivec)
    e = jnp.exp(vvec - jnp.max(vvec))
    scores = e / jnp.sum(e)
    score_ref[...] = scores.reshape(1, 1, _TOPK)
    idx_ref[...] = ivec.reshape(1, 1, _TOPK)


def kernel(heatmap):
    bs, nc, h, w = heatmap.shape
    g = bs * nc
    hm = heatmap.reshape(g, h, w)
    scores, idxs = pl.pallas_call(
        _peaks_kernel,
        grid=(g,),
        in_specs=[pl.BlockSpec((1, h, w), lambda i: (i, 0, 0))],
        out_specs=[
            pl.BlockSpec((1, 1, _TOPK), lambda i: (i, 0, 0)),
            pl.BlockSpec((1, 1, _TOPK), lambda i: (i, 0, 0)),
        ],
        out_shape=[
            jax.ShapeDtypeStruct((g, 1, _TOPK), jnp.float32),
            jax.ShapeDtypeStruct((g, 1, _TOPK), jnp.int32),
        ],
    )(hm)
    idxs = idxs.reshape(bs, nc, _TOPK)
    scores = scores.reshape(bs, nc, _TOPK)
    coords = jnp.stack([idxs // w, idxs % w], axis=-1)
    return (coords, scores, idxs)


# speedup vs baseline: 3.2684x; 2.4921x over previous
import jax
import jax.numpy as jnp
from jax.experimental import pallas as pl
from jax.experimental.pallas import tpu as pltpu

_TOPK = 8
_PAD = 2
_SUB = 8
_STRIP = 64
_NMAPS = 4


def _hshift(x, s, fill):
    h, w = x.shape
    pad = jnp.full((h, abs(s)), fill, x.dtype)
    if s > 0:
        return jnp.concatenate([x[:, s:], pad], axis=1)
    return jnp.concatenate([pad, x[:, :w + s]], axis=1)


def _hmax5(v):
    ninf = jnp.float32(-jnp.inf)
    p = v
    for s in (-2, -1, 1, 2):
        p = jnp.maximum(p, _hshift(v, s, ninf))
    return p


def _sub_bcast(x, op):
    for s in (32, 16, 8, 4, 2, 1):
        x = op(x, pltpu.roll(x, s, axis=0))
    return x


def _strip_pass(x_ref, mm_ref, m, s, h, w):
    ninf = jnp.float32(-jnp.inf)
    n_sub = _STRIP // _SUB
    r0 = s * _STRIP
    if r0 - _PAD >= 0 and r0 + _STRIP + _PAD <= h:
        v = None
        for d in range(-_PAD, _PAD + 1):
            xd = x_ref[m, r0 + d:r0 + d + _STRIP, :]
            v = xd if v is None else jnp.maximum(v, xd)
    else:
        lo = max(0, r0 - _PAD)
        hi = min(h, r0 + _STRIP + _PAD)
        parts = []
        if r0 - _PAD < 0:
            parts.append(jnp.full((_PAD - r0, w), ninf, jnp.float32))
        parts.append(x_ref[m, lo:hi, :])
        if r0 + _STRIP + _PAD > h:
            parts.append(jnp.full((r0 + _STRIP + _PAD - h, w), ninf,
                                  jnp.float32))
        xp = jnp.concatenate(parts, axis=0)
        v = xp[0:_STRIP]
        for d in range(1, 5):
            v = jnp.maximum(v, xp[d:d + _STRIP])
    p = _hmax5(v)
    xc = x_ref[m, r0:r0 + _STRIP, :]
    mm = jnp.where(xc == p, xc, jnp.float32(0.0))
    mm_ref[r0:r0 + _STRIP, :] = mm
    t = jnp.max(mm.reshape(n_sub, _SUB, w), axis=1)
    tmax = jnp.max(t, axis=1)
    return tmax.reshape(n_sub, 1)


def _extract_round(mm_ref, st):
    smax, vals, idxs, iota64, lin_i0, lin_i1, w = st
    ninf = jnp.float32(-jnp.inf)
    big = jnp.int32(2**30)
    gmb = _sub_bcast(smax, jnp.maximum)
    sib = _sub_bcast(jnp.where(smax == gmb, iota64, big), jnp.minimum)
    si = sib[0, 0]
    gm = gmb[0, 0]
    r0 = si * _SUB
    mm = mm_ref[pl.ds(r0, _SUB), :]
    lin = (lin_i0 + r0) * w + lin_i1
    eq = mm == gm
    gi = jnp.min(jnp.where(eq, lin, big))
    exm = jnp.max(jnp.where(eq, ninf, mm))
    cnt = jnp.sum(jnp.where(eq, jnp.float32(1.0), jnp.float32(0.0)))
    m2 = jnp.where(lin == gi, ninf, mm)
    mm_ref[pl.ds(r0, _SUB), :] = m2
    nsm = jnp.where(cnt >= 1.5, gm, exm)
    smax = jnp.where(iota64 == si, nsm, smax)
    vals.append(gm)
    idxs.append(gi)
    return (smax, vals, idxs, iota64, lin_i0, lin_i1, w)


def _peaks_kernel(x_ref, score_ref, idx_ref, *mm_refs):
    nm, h, w = x_ref.shape
    n_strips = h // _STRIP
    n_sub = _STRIP // _SUB
    ninf = jnp.float32(-jnp.inf)
    iota64 = jax.lax.broadcasted_iota(jnp.int32, (n_strips * n_sub, 1), 0)
    lin_i0 = jax.lax.broadcasted_iota(jnp.int32, (_SUB, w), 0)
    lin_i1 = jax.lax.broadcasted_iota(jnp.int32, (_SUB, w), 1)
    k_iota = jax.lax.broadcasted_iota(jnp.int32, (1, _TOPK), 1)

    def main_pass(m):
        parts = [_strip_pass(x_ref, mm_refs[m], m, s, h, w)
                 for s in range(n_strips)]
        smax = jnp.concatenate(parts, axis=0)
        return (smax, [], [], iota64, lin_i0, lin_i1, w)

    def finish(m, st):
        _, vals, idxs, *_ = st
        vvec = jnp.full((1, _TOPK), jnp.float32(0.0))
        ivec = jnp.zeros((1, _TOPK), jnp.int32)
        for k in range(_TOPK):
            vvec = jnp.where(k_iota == k, vals[k], vvec)
            ivec = jnp.where(k_iota == k, idxs[k], ivec)
        vmax = jnp.max(vvec)
        e = jnp.exp(vvec - vmax)
        score_ref[m] = (e / jnp.sum(e)).reshape(1, _TOPK)
        idx_ref[m] = ivec.reshape(1, _TOPK)

    st = main_pass(0)
    for m in range(1, nm):
        parts = []
        for s in range(n_strips):
            parts.append(_strip_pass(x_ref, mm_refs[m], m, s, h, w))
            if s < _TOPK:
                st = _extract_round(mm_refs[m - 1], st)
        for _ in range(len(st[1]), _TOPK):
            st = _extract_round(mm_refs[m - 1], st)
        finish(m - 1, st)
        st = (jnp.concatenate(parts, axis=0), [], [], iota64, lin_i0,
              lin_i1, w)
    for _ in range(_TOPK):
        st = _extract_round(mm_refs[nm - 1], st)
    finish(nm - 1, st)


def kernel(heatmap):
    bs, nc, h, w = heatmap.shape
    g = bs * nc
    nm = _NMAPS if g % _NMAPS == 0 else 1
    hm = heatmap.reshape(g, h, w)
    scores, idxs = pl.pallas_call(
        _peaks_kernel,
        grid=(g // nm,),
        in_specs=[pl.BlockSpec((nm, h, w), lambda i: (i, 0, 0))],
        out_specs=[
            pl.BlockSpec((nm, 1, _TOPK), lambda i: (i, 0, 0)),
            pl.BlockSpec((nm, 1, _TOPK), lambda i: (i, 0, 0)),
        ],
        out_shape=[
            jax.ShapeDtypeStruct((g, 1, _TOPK), jnp.float32),
            jax.ShapeDtypeStruct((g, 1, _TOPK), jnp.int32),
        ],
        scratch_shapes=[pltpu.VMEM((h, w), jnp.float32) for _ in range(nm)],
    )(hm)
    idxs = idxs.reshape(bs, nc, _TOPK)
    scores = scores.reshape(bs, nc, _TOPK)
    coords = jnp.stack([idxs // w, idxs % w], axis=-1)
    return (coords, scores, idxs)


# speedup vs baseline: 3.6484x; 1.1163x over previous
import jax
import jax.numpy as jnp
from jax.experimental import pallas as pl
from jax.experimental.pallas import tpu as pltpu

_TOPK = 8
_PAD = 2
_SUB = 8
_STRIP = 64
_NMAPS = 8


def _hshift(x, s, fill):
    h, w = x.shape
    pad = jnp.full((h, abs(s)), fill, x.dtype)
    if s > 0:
        return jnp.concatenate([x[:, s:], pad], axis=1)
    return jnp.concatenate([pad, x[:, :w + s]], axis=1)


def _hmax5(v):
    ninf = jnp.float32(-jnp.inf)
    p = v
    for s in (-2, -1, 1, 2):
        p = jnp.maximum(p, _hshift(v, s, ninf))
    return p


def _sub_bcast(x, op):
    for s in (32, 16, 8, 4, 2, 1):
        x = op(x, pltpu.roll(x, s, axis=0))
    return x


def _strip_pass(x_ref, mm_ref, m, s, h, w):
    ninf = jnp.float32(-jnp.inf)
    n_sub = _STRIP // _SUB
    r0 = s * _STRIP
    if r0 - _PAD >= 0 and r0 + _STRIP + _PAD <= h:
        v = None
        for d in range(-_PAD, _PAD + 1):
            xd = x_ref[m, r0 + d:r0 + d + _STRIP, :]
            v = xd if v is None else jnp.maximum(v, xd)
    else:
        lo = max(0, r0 - _PAD)
        hi = min(h, r0 + _STRIP + _PAD)
        parts = []
        if r0 - _PAD < 0:
            parts.append(jnp.full((_PAD - r0, w), ninf, jnp.float32))
        parts.append(x_ref[m, lo:hi, :])
        if r0 + _STRIP + _PAD > h:
            parts.append(jnp.full((r0 + _STRIP + _PAD - h, w), ninf,
                                  jnp.float32))
        xp = jnp.concatenate(parts, axis=0)
        v = xp[0:_STRIP]
        for d in range(1, 5):
            v = jnp.maximum(v, xp[d:d + _STRIP])
    p = _hmax5(v)
    xc = x_ref[m, r0:r0 + _STRIP, :]
    mm = jnp.where(xc == p, xc, jnp.float32(0.0))
    mm_ref[r0:r0 + _STRIP, :] = mm
    t = jnp.max(mm.reshape(n_sub, _SUB, w), axis=1)
    tmax = jnp.max(t, axis=1)
    return tmax.reshape(n_sub, 1)


def _extract_round(mm_ref, st):
    smax, vals, idxs, iota64, lin_i0, lin_i1, w = st
    ninf = jnp.float32(-jnp.inf)
    big = jnp.int32(2**30)
    gmb = _sub_bcast(smax, jnp.maximum)
    sib = _sub_bcast(jnp.where(smax == gmb, iota64, big), jnp.minimum)
    si = sib[0, 0]
    gm = gmb[0, 0]
    r0 = si * _SUB
    mm = mm_ref[pl.ds(r0, _SUB), :]
    lin = (lin_i0 + r0) * w + lin_i1
    eq = mm == gm
    gi = jnp.min(jnp.where(eq, lin, big))
    exm = jnp.max(jnp.where(eq, ninf, mm))
    cnt = jnp.sum(jnp.where(eq, jnp.float32(1.0), jnp.float32(0.0)))
    m2 = jnp.where(lin == gi, ninf, mm)
    mm_ref[pl.ds(r0, _SUB), :] = m2
    nsm = jnp.where(cnt >= 1.5, gm, exm)
    smax = jnp.where(iota64 == si, nsm, smax)
    vals.append(gm)
    idxs.append(gi)
    return (smax, vals, idxs, iota64, lin_i0, lin_i1, w)


def _peaks_kernel(x_ref, score_ref, idx_ref, *mm_refs):
    nm, h, w = x_ref.shape
    n_strips = h // _STRIP
    n_sub = _STRIP // _SUB
    ninf = jnp.float32(-jnp.inf)
    iota64 = jax.lax.broadcasted_iota(jnp.int32, (n_strips * n_sub, 1), 0)
    lin_i0 = jax.lax.broadcasted_iota(jnp.int32, (_SUB, w), 0)
    lin_i1 = jax.lax.broadcasted_iota(jnp.int32, (_SUB, w), 1)
    k_iota = jax.lax.broadcasted_iota(jnp.int32, (1, _TOPK), 1)

    def main_pass(m):
        parts = [_strip_pass(x_ref, mm_refs[m], m, s, h, w)
                 for s in range(n_strips)]
        smax = jnp.concatenate(parts, axis=0)
        return (smax, [], [], iota64, lin_i0, lin_i1, w)

    def finish(m, st):
        _, vals, idxs, *_ = st
        vvec = jnp.full((1, _TOPK), jnp.float32(0.0))
        ivec = jnp.zeros((1, _TOPK), jnp.int32)
        for k in range(_TOPK):
            vvec = jnp.where(k_iota == k, vals[k], vvec)
            ivec = jnp.where(k_iota == k, idxs[k], ivec)
        vmax = jnp.max(vvec)
        e = jnp.exp(vvec - vmax)
        score_ref[m] = (e / jnp.sum(e)).reshape(1, _TOPK)
        idx_ref[m] = ivec.reshape(1, _TOPK)

    st = main_pass(0)
    for m in range(1, nm):
        parts = []
        for s in range(n_strips):
            parts.append(_strip_pass(x_ref, mm_refs[m], m, s, h, w))
            if s < _TOPK:
                st = _extract_round(mm_refs[m - 1], st)
        for _ in range(len(st[1]), _TOPK):
            st = _extract_round(mm_refs[m - 1], st)
        finish(m - 1, st)
        st = (jnp.concatenate(parts, axis=0), [], [], iota64, lin_i0,
              lin_i1, w)
    for _ in range(_TOPK):
        st = _extract_round(mm_refs[nm - 1], st)
    finish(nm - 1, st)


def kernel(heatmap):
    bs, nc, h, w = heatmap.shape
    g = bs * nc
    nm = _NMAPS if g % _NMAPS == 0 else 1
    hm = heatmap.reshape(g, h, w)
    scores, idxs = pl.pallas_call(
        _peaks_kernel,
        grid=(g // nm,),
        in_specs=[pl.BlockSpec((nm, h, w), lambda i: (i, 0, 0))],
        out_specs=[
            pl.BlockSpec((nm, 1, _TOPK), lambda i: (i, 0, 0)),
            pl.BlockSpec((nm, 1, _TOPK), lambda i: (i, 0, 0)),
        ],
        out_shape=[
            jax.ShapeDtypeStruct((g, 1, _TOPK), jnp.float32),
            jax.ShapeDtypeStruct((g, 1, _TOPK), jnp.int32),
        ],
        scratch_shapes=[pltpu.VMEM((h, w), jnp.float32) for _ in range(nm)],
    )(hm)
    idxs = idxs.reshape(bs, nc, _TOPK)
    scores = scores.reshape(bs, nc, _TOPK)
    coords = jnp.stack([idxs // w, idxs % w], axis=-1)
    return (coords, scores, idxs)


# speedup vs baseline: 3.6504x; 1.0005x over previous
import jax
import jax.numpy as jnp
from jax.experimental import pallas as pl
from jax.experimental.pallas import tpu as pltpu

_TOPK = 8
_PAD = 2
_SUB = 8
_STRIP = 64
_NMAPS = 8


def _hshift(x, s, fill):
    h, w = x.shape
    pad = jnp.full((h, abs(s)), fill, x.dtype)
    if s > 0:
        return jnp.concatenate([x[:, s:], pad], axis=1)
    return jnp.concatenate([pad, x[:, :w + s]], axis=1)


def _hmax5(v):
    ninf = jnp.float32(-jnp.inf)
    p = v
    for s in (-2, -1, 1, 2):
        p = jnp.maximum(p, _hshift(v, s, ninf))
    return p


def _sub_bcast(x, op):
    for s in (32, 16, 8, 4, 2, 1):
        x = op(x, pltpu.roll(x, s, axis=0))
    return x


def _strip_pass(x_ref, mm_ref, m, s, h, w):
    ninf = jnp.float32(-jnp.inf)
    n_sub = _STRIP // _SUB
    r0 = s * _STRIP
    if r0 - _PAD >= 0 and r0 + _STRIP + _PAD <= h:
        v = None
        for d in range(-_PAD, _PAD + 1):
            xd = x_ref[m, r0 + d:r0 + d + _STRIP, :]
            v = xd if v is None else jnp.maximum(v, xd)
    else:
        lo = max(0, r0 - _PAD)
        hi = min(h, r0 + _STRIP + _PAD)
        parts = []
        if r0 - _PAD < 0:
            parts.append(jnp.full((_PAD - r0, w), ninf, jnp.float32))
        parts.append(x_ref[m, lo:hi, :])
        if r0 + _STRIP + _PAD > h:
            parts.append(jnp.full((r0 + _STRIP + _PAD - h, w), ninf,
                                  jnp.float32))
        xp = jnp.concatenate(parts, axis=0)
        v = xp[0:_STRIP]
        for d in range(1, 5):
            v = jnp.maximum(v, xp[d:d + _STRIP])
    p = _hmax5(v)
    xc = x_ref[m, r0:r0 + _STRIP, :]
    mm = jnp.where(xc == p, xc, jnp.float32(0.0))
    mm_ref[r0:r0 + _STRIP, :] = mm
    t = jnp.max(mm.reshape(n_sub, _SUB, w), axis=1)
    tmax = jnp.max(t, axis=1)
    return tmax.reshape(n_sub, 1)


def _extract_round(mm_ref, st):
    smax, vals, idxs, iota64, lin_i0, lin_i1, w = st
    ninf = jnp.float32(-jnp.inf)
    big = jnp.int32(2**30)
    gmb = _sub_bcast(smax, jnp.maximum)
    sib = _sub_bcast(jnp.where(smax == gmb, iota64, big), jnp.minimum)
    si = sib[0, 0]
    gm = gmb[0, 0]
    r0 = si * _SUB
    mm = mm_ref[pl.ds(r0, _SUB), :]
    lin = (lin_i0 + r0) * w + lin_i1
    eq = mm == gm
    gi = jnp.min(jnp.where(eq, lin, big))
    exm = jnp.max(jnp.where(eq, ninf, mm))
    cnt = jnp.sum(jnp.where(eq, jnp.float32(1.0), jnp.float32(0.0)))
    m2 = jnp.where(lin == gi, ninf, mm)
    mm_ref[pl.ds(r0, _SUB), :] = m2
    nsm = jnp.where(cnt >= 1.5, gm, exm)
    smax = jnp.where(iota64 == si, nsm, smax)
    vals.append(gm)
    idxs.append(gi)
    return (smax, vals, idxs, iota64, lin_i0, lin_i1, w)


def _peaks_kernel(x_ref, score_ref, idx_ref, *mm_refs):
    nm, h, w = x_ref.shape
    n_strips = h // _STRIP
    n_sub = _STRIP // _SUB
    iota64 = jax.lax.broadcasted_iota(jnp.int32, (n_strips * n_sub, 1), 0)
    lin_i0 = jax.lax.broadcasted_iota(jnp.int32, (_SUB, w), 0)
    lin_i1 = jax.lax.broadcasted_iota(jnp.int32, (_SUB, w), 1)
    k_iota = jax.lax.broadcasted_iota(jnp.int32, (1, _TOPK), 1)

    def main_pass(m):
        parts = [_strip_pass(x_ref, mm_refs[m], m, s, h, w)
                 for s in range(n_strips)]
        smax = jnp.concatenate(parts, axis=0)
        return (smax, [], [], iota64, lin_i0, lin_i1, w)

    def finish(m, st):
        _, vals, idxs, *_ = st
        vvec = jnp.full((1, _TOPK), jnp.float32(0.0))
        ivec = jnp.zeros((1, _TOPK), jnp.int32)
        for k in range(_TOPK):
            vvec = jnp.where(k_iota == k, vals[k], vvec)
            ivec = jnp.where(k_iota == k, idxs[k], ivec)
        vmax = jnp.max(vvec)
        e = jnp.exp(vvec - vmax)
        score_ref[m] = (e / jnp.sum(e)).reshape(1, _TOPK)
        idx_ref[m] = ivec.reshape(1, _TOPK)

    st = main_pass(0)
    for m in range(1, nm):
        parts = []
        for s in range(n_strips):
            parts.append(_strip_pass(x_ref, mm_refs[m], m, s, h, w))
            while len(st[1]) < (s + 1) * _TOPK // n_strips:
                st = _extract_round(mm_refs[m - 1], st)
        for _ in range(len(st[1]), _TOPK):
            st = _extract_round(mm_refs[m - 1], st)
        finish(m - 1, st)
        st = (jnp.concatenate(parts, axis=0), [], [], iota64, lin_i0,
              lin_i1, w)
    for _ in range(_TOPK):
        st = _extract_round(mm_refs[nm - 1], st)
    finish(nm - 1, st)


def kernel(heatmap):
    bs, nc, h, w = heatmap.shape
    g = bs * nc
    nm = _NMAPS if g % _NMAPS == 0 else 1
    hm = heatmap.reshape(g, h, w)
    scores, idxs = pl.pallas_call(
        _peaks_kernel,
        grid=(g // nm,),
        in_specs=[pl.BlockSpec((nm, h, w), lambda i: (i, 0, 0))],
        out_specs=[
            pl.BlockSpec((nm, 1, _TOPK), lambda i: (i, 0, 0)),
            pl.BlockSpec((nm, 1, _TOPK), lambda i: (i, 0, 0)),
        ],
        out_shape=[
            jax.ShapeDtypeStruct((g, 1, _TOPK), jnp.float32),
            jax.ShapeDtypeStruct((g, 1, _TOPK), jnp.int32),
        ],
        scratch_shapes=[pltpu.VMEM((h, w), jnp.float32) for _ in range(nm)],
    )(hm)
    idxs = idxs.reshape(bs, nc, _TOPK)
    scores = scores.reshape(bs, nc, _TOPK)
    coords = jnp.stack([idxs // w, idxs % w], axis=-1)
    return (coords, scores, idxs)
